# Initial kernel scaffold; baseline (speedup 1.0000x reference)
import jax
import jax.numpy as jnp
from jax.experimental import pallas as pl


def kernel(x, edge_index, W):
    raise NotImplementedError("write your pallas kernel here")


# speedup vs baseline: 4.3727x; 4.3727x over previous
import functools

import jax
import jax.numpy as jnp
from jax import lax
from jax.experimental import pallas as pl
from jax.experimental.pallas import tpu as pltpu
from jax.experimental.pallas import tpu_sc as plsc

N_NODES = 10000
N_EDGES = 160000
DIM_IN = 256
DIM_HALF = 128
NUM_CORES = 2
NUM_SUBCORES = 16
CHUNK = 128
N_CHUNKS = N_EDGES // CHUNK
FULL_ROUNDS = N_CHUNKS // NUM_SUBCORES
TAIL = N_CHUNKS - FULL_ROUNDS * NUM_SUBCORES
ROWS_PER_TILE = 624
ROWS_REM = N_NODES - ROWS_PER_TILE * NUM_SUBCORES


def _mm_body(x_ref, w_ref, o_ref):
    o_ref[...] = lax.dot_general(
        x_ref[...], w_ref[...], (((1,), (1,)), ((), ())),
        preferred_element_type=jnp.float32)


def _matmul_split(x, W):
    m_blk = 1000
    grid = (N_NODES // m_blk, NUM_CORES)
    return pl.pallas_call(
        _mm_body,
        grid=grid,
        in_specs=[
            pl.BlockSpec((m_blk, DIM_IN), lambda i, c: (i, 0)),
            pl.BlockSpec((DIM_HALF, DIM_IN), lambda i, c: (c, 0)),
        ],
        out_specs=pl.BlockSpec(
            (m_blk, DIM_HALF),
            lambda i, c: (c * (N_NODES // m_blk) + i, 0)),
        out_shape=jax.ShapeDtypeStruct((NUM_CORES * N_NODES, DIM_HALF),
                                       jnp.float32),
    )(x, W)


def _sc_aggregate(h2, src, dst, zeros):
    mesh = plsc.VectorSubcoreMesh(
        core_axis_name="c", subcore_axis_name="s",
        num_cores=NUM_CORES, num_subcores=NUM_SUBCORES)

    @functools.partial(
        pl.kernel,
        out_type=jax.ShapeDtypeStruct((NUM_CORES * N_NODES, DIM_HALF),
                                      jnp.float32),
        mesh=mesh,
        scratch_types=[
            pltpu.VMEM((CHUNK,), jnp.int32),
            pltpu.VMEM((CHUNK,), jnp.int32),
            pltpu.VMEM((CHUNK, DIM_HALF), jnp.float32),
            pltpu.VMEM_SHARED((N_NODES, DIM_HALF), jnp.float32),
            pltpu.SemaphoreType.DMA,
        ],
    )
    def agg(h_hbm, src_hbm, dst_hbm, z_hbm, out_hbm,
            sidx, didx, rows, acc, sem):
        c = lax.axis_index("c")
        s = lax.axis_index("s")
        row0 = s * ROWS_PER_TILE
        pltpu.sync_copy(z_hbm.at[pl.ds(0, ROWS_PER_TILE)],
                        acc.at[pl.ds(row0, ROWS_PER_TILE)])

        @pl.when(s == NUM_SUBCORES - 1)
        def _():
            pltpu.sync_copy(
                z_hbm.at[pl.ds(0, ROWS_REM)],
                acc.at[pl.ds(ROWS_PER_TILE * NUM_SUBCORES, ROWS_REM)])

        plsc.subcore_barrier()

        off = c * N_NODES

        def process(ci):
            base = ci * CHUNK
            pltpu.sync_copy(src_hbm.at[pl.ds(base, CHUNK)], sidx)

            @pl.loop(0, CHUNK, step=16)
            def _(k):
                sidx[pl.ds(k, 16)] = sidx[pl.ds(k, 16)] + off

            pltpu.async_copy(h_hbm.at[sidx], rows, sem).wait()
            pltpu.sync_copy(dst_hbm.at[pl.ds(base, CHUNK)], didx)
            pltpu.sync_copy(rows, acc.at[didx], add=True)

        @pl.loop(0, FULL_ROUNDS)
        def _(j):
            process(j * NUM_SUBCORES + s)

        @pl.when(s < TAIL)
        def _():
            process(FULL_ROUNDS * NUM_SUBCORES + s)

        plsc.subcore_barrier()
        pltpu.sync_copy(acc.at[pl.ds(row0, ROWS_PER_TILE)],
                        out_hbm.at[pl.ds(c * N_NODES + row0, ROWS_PER_TILE)])

        @pl.when(s == NUM_SUBCORES - 1)
        def _():
            tail0 = ROWS_PER_TILE * NUM_SUBCORES
            pltpu.sync_copy(acc.at[pl.ds(tail0, ROWS_REM)],
                            out_hbm.at[pl.ds(c * N_NODES + tail0, ROWS_REM)])

    return agg(h2, src, dst, zeros)


def kernel(x, edge_index, W):
    src = edge_index[0].astype(jnp.int32)
    dst = edge_index[1].astype(jnp.int32)
    h2 = _matmul_split(x, W)
    zeros = jnp.zeros((ROWS_PER_TILE + ROWS_REM, DIM_HALF), jnp.float32)
    out2 = _sc_aggregate(h2, src, dst, zeros)
    return jnp.concatenate([out2[:N_NODES], out2[N_NODES:]], axis=1)
